# baseline (speedup 1.0000x reference)
import jax
import jax.numpy as jnp
from jax import lax
from jax.experimental import pallas as pl
from jax.experimental.pallas import tpu as pltpu
from jax.experimental.pallas import tpu_sc as plsc

VOCAB = 100000
D = 128
B = 4096
NS = 5
NSC = 6
NIDX = 7
NC = 2
NSUB = 16
NW = NC * NSUB
BPW = B // NW
L = 16
KCH = D // L
NG = BPW // L
PPITCH = BPW + 1


def _sg_body(idx_hbm, emb_hbm, ctx_hbm, out_hbm,
             idx_v, crows, prows, nrows, part, sall,
             sem_i, sem_w0, sem_w1):
    wid = lax.axis_index("s") * NC + lax.axis_index("c")

    pltpu.async_copy(idx_hbm.at[wid], idx_v, sem_i).wait()

    def gather(tab, slot, dst, sem):
        return pltpu.async_copy(tab.at[idx_v.at[slot]], dst, sem)

    w0 = [gather(emb_hbm, 0, crows, sem_w0),
          gather(ctx_hbm, 1, prows, sem_w0),
          gather(ctx_hbm, 2, nrows.at[0], sem_w0),
          gather(ctx_hbm, 3, nrows.at[1], sem_w0)]
    w1 = [gather(ctx_hbm, 4, nrows.at[2], sem_w1),
          gather(ctx_hbm, 5, nrows.at[3], sem_w1),
          gather(ctx_hbm, 6, nrows.at[4], sem_w1)]

    lane = lax.iota(jnp.int32, L)

    def scatter_part(i, b, acc):
        idx = lane * PPITCH + (b + (i * L * PPITCH))
        plsc.store_scatter(part, [idx], acc)

    def run_group(srcs_refs, s0):
        def body(b, carry):
            c = [crows[b, pl.ds(k * L, L)] for k in range(KCH)]
            for i, r in enumerate(srcs_refs):
                acc = c[0] * r[b, pl.ds(0, L)]
                for k in range(1, KCH):
                    acc = acc + c[k] * r[b, pl.ds(k * L, L)]
                scatter_part(i, b, acc)
            return carry

        lax.fori_loop(0, BPW, body, 0, unroll=2)
        for i in range(len(srcs_refs)):
            for g in range(NG):
                acc = part[pl.ds(i * L * PPITCH + g * L, L)]
                for t in range(1, L):
                    acc = acc + part[pl.ds((i * L + t) * PPITCH + g * L, L)]
                sall[s0 + i, pl.ds(g * L, L)] = acc

    for cp in w0:
        cp.wait()
    run_group([prows, nrows.at[0], nrows.at[1]], 0)
    for cp in w1:
        cp.wait()
    run_group([nrows.at[2], nrows.at[3], nrows.at[4]], 3)

    pltpu.sync_copy(sall, out_hbm.at[wid])


@jax.jit
def _skipgram(idx_all, embeddings, context_embeddings):
    mesh = plsc.VectorSubcoreMesh(
        core_axis_name="c", subcore_axis_name="s",
        num_cores=NC, num_subcores=NSUB)
    return pl.kernel(
        _sg_body,
        out_type=jax.ShapeDtypeStruct((NW, NSC, BPW), jnp.float32),
        mesh=mesh,
        compiler_params=pltpu.CompilerParams(needs_layout_passes=False),
        scratch_types=[
            pltpu.VMEM((NIDX, BPW), jnp.int32),
            pltpu.VMEM((BPW, D), jnp.float32),
            pltpu.VMEM((BPW, D), jnp.float32),
            pltpu.VMEM((NS, BPW, D), jnp.float32),
            pltpu.VMEM((3 * L * PPITCH,), jnp.float32),
            pltpu.VMEM((NSC, BPW), jnp.float32),
            pltpu.SemaphoreType.DMA,
            pltpu.SemaphoreType.DMA,
            pltpu.SemaphoreType.DMA,
        ],
    )(idx_all, embeddings, context_embeddings)


def kernel(center_word, context_word, negative_samples, embeddings, context_embeddings):
    idx_all = jnp.concatenate(
        [center_word[None, :], context_word[None, :], negative_samples.T],
        axis=0)
    idx_all = idx_all.reshape(NIDX, NW, BPW).transpose(1, 0, 2)
    out = _skipgram(idx_all, embeddings, context_embeddings)
    pos_score = out[:, 0, :].reshape(B)
    neg_score = out[:, 1:, :].transpose(0, 2, 1).reshape(B, NS)
    return (pos_score, neg_score)


# speedup vs baseline: 1.0163x; 1.0163x over previous
import jax
import jax.numpy as jnp
from jax import lax
from jax.experimental import pallas as pl
from jax.experimental.pallas import tpu as pltpu
from jax.experimental.pallas import tpu_sc as plsc

VOCAB = 100000
D = 128
B = 4096
NS = 5
NSC = 6
NIDX = 7
NC = 2
NSUB = 16
NW = NC * NSUB
BPW = B // NW
L = 16
KCH = D // L
NG = BPW // L
PPITCH = BPW + 1


def _sg_body(idx_hbm, emb_hbm, ctx_hbm, out_hbm,
             idx_v, crows, prows, nrows, part, sall,
             sem_i, sem_w0, sem_w1, sem_w2):
    wid = lax.axis_index("s") * NC + lax.axis_index("c")

    pltpu.async_copy(idx_hbm.at[wid], idx_v, sem_i).wait()

    def gather(tab, slot, dst, sem):
        return pltpu.async_copy(tab.at[idx_v.at[slot]], dst, sem)

    w0 = [gather(emb_hbm, 0, crows, sem_w0),
          gather(ctx_hbm, 1, prows, sem_w0)]
    w1 = [gather(ctx_hbm, 2, nrows.at[0], sem_w1),
          gather(ctx_hbm, 3, nrows.at[1], sem_w1),
          gather(ctx_hbm, 4, nrows.at[2], sem_w1)]
    w2 = [gather(ctx_hbm, 5, nrows.at[3], sem_w2),
          gather(ctx_hbm, 6, nrows.at[4], sem_w2)]

    lane = lax.iota(jnp.int32, L)

    def scatter_part(i, b, acc):
        idx = lane * PPITCH + (b + (i * L * PPITCH))
        plsc.store_scatter(part, [idx], acc)

    def run_group(srcs_refs, s0):
        def body(b, carry):
            c = [crows[b, pl.ds(k * L, L)] for k in range(KCH)]
            for i, r in enumerate(srcs_refs):
                acc = c[0] * r[b, pl.ds(0, L)]
                for k in range(1, KCH):
                    acc = acc + c[k] * r[b, pl.ds(k * L, L)]
                scatter_part(i, b, acc)
            return carry

        lax.fori_loop(0, BPW, body, 0, unroll=2)
        for i in range(len(srcs_refs)):
            for g in range(NG):
                acc = part[pl.ds(i * L * PPITCH + g * L, L)]
                for t in range(1, L):
                    acc = acc + part[pl.ds((i * L + t) * PPITCH + g * L, L)]
                sall[s0 + i, pl.ds(g * L, L)] = acc

    for cp in w0:
        cp.wait()
    run_group([prows], 0)
    for cp in w1:
        cp.wait()
    run_group([nrows.at[0], nrows.at[1], nrows.at[2]], 1)
    for cp in w2:
        cp.wait()
    run_group([nrows.at[3], nrows.at[4]], 4)

    pltpu.sync_copy(sall, out_hbm.at[wid])


@jax.jit
def _skipgram(idx_all, embeddings, context_embeddings):
    mesh = plsc.VectorSubcoreMesh(
        core_axis_name="c", subcore_axis_name="s",
        num_cores=NC, num_subcores=NSUB)
    return pl.kernel(
        _sg_body,
        out_type=jax.ShapeDtypeStruct((NW, NSC, BPW), jnp.float32),
        mesh=mesh,
        compiler_params=pltpu.CompilerParams(needs_layout_passes=False),
        scratch_types=[
            pltpu.VMEM((NIDX, BPW), jnp.int32),
            pltpu.VMEM((BPW, D), jnp.float32),
            pltpu.VMEM((BPW, D), jnp.float32),
            pltpu.VMEM((NS, BPW, D), jnp.float32),
            pltpu.VMEM((3 * L * PPITCH,), jnp.float32),
            pltpu.VMEM((NSC, BPW), jnp.float32),
            pltpu.SemaphoreType.DMA,
            pltpu.SemaphoreType.DMA,
            pltpu.SemaphoreType.DMA,
            pltpu.SemaphoreType.DMA,
        ],
    )(idx_all, embeddings, context_embeddings)


def kernel(center_word, context_word, negative_samples, embeddings, context_embeddings):
    idx_all = jnp.concatenate(
        [center_word[None, :], context_word[None, :], negative_samples.T],
        axis=0)
    idx_all = idx_all.reshape(NIDX, NW, BPW).transpose(1, 0, 2)
    out = _skipgram(idx_all, embeddings, context_embeddings)
    pos_score = out[:, 0, :].reshape(B)
    neg_score = out[:, 1:, :].transpose(0, 2, 1).reshape(B, NS)
    return (pos_score, neg_score)


# speedup vs baseline: 1.1528x; 1.1344x over previous
import jax
import jax.numpy as jnp
from jax import lax
from jax.experimental import pallas as pl
from jax.experimental.pallas import tpu as pltpu
from jax.experimental.pallas import tpu_sc as plsc

VOCAB = 100000
D = 128
B = 4096
NS = 5
NSC = 6
NIDX = 7
NC = 2
NSUB = 16
NW = NC * NSUB
BPW = B // NW
L = 16
KCH = D // L
NG = BPW // L
PPITCH = BPW + 1


def _sg_body(idx_hbm, emb_hbm, ctx_hbm, out_hbm,
             idx_v, crows, prows, nrows, part, sall,
             sem_i, sem_w0, sem_w1, sem_w2):
    wid = lax.axis_index("s") * NC + lax.axis_index("c")

    pltpu.async_copy(idx_hbm.at[wid], idx_v, sem_i).wait()

    def gather(tab, slot, dst, sem):
        return pltpu.async_copy(tab.at[idx_v.at[slot]], dst, sem)

    w0 = [gather(emb_hbm, 0, crows, sem_w0),
          gather(ctx_hbm, 1, prows, sem_w0)]
    w1 = [gather(ctx_hbm, 2, nrows.at[0], sem_w1),
          gather(ctx_hbm, 3, nrows.at[1], sem_w1),
          gather(ctx_hbm, 4, nrows.at[2], sem_w1)]
    w2 = [gather(ctx_hbm, 5, nrows.at[3], sem_w2),
          gather(ctx_hbm, 6, nrows.at[4], sem_w2)]

    lane = lax.iota(jnp.int32, L)

    def scatter_part(i, b, acc):
        idx = lane * PPITCH + (b + (i * L * PPITCH))
        plsc.store_scatter(part, [idx], acc)

    def run_group(srcs_refs, s0):
        @plsc.parallel_loop(0, BPW, step=1, unroll=2)
        def _(b):
            c = [crows[b, pl.ds(k * L, L)] for k in range(KCH)]
            for i, r in enumerate(srcs_refs):
                acc = c[0] * r[b, pl.ds(0, L)]
                for k in range(1, KCH):
                    acc = acc + c[k] * r[b, pl.ds(k * L, L)]
                scatter_part(i, b, acc)
        for i in range(len(srcs_refs)):
            for g in range(NG):
                acc = part[pl.ds(i * L * PPITCH + g * L, L)]
                for t in range(1, L):
                    acc = acc + part[pl.ds((i * L + t) * PPITCH + g * L, L)]
                sall[s0 + i, pl.ds(g * L, L)] = acc

    for cp in w0:
        cp.wait()
    run_group([prows], 0)
    for cp in w1:
        cp.wait()
    run_group([nrows.at[0], nrows.at[1], nrows.at[2]], 1)
    for cp in w2:
        cp.wait()
    run_group([nrows.at[3], nrows.at[4]], 4)

    pltpu.sync_copy(sall, out_hbm.at[wid])


@jax.jit
def _skipgram(idx_all, embeddings, context_embeddings):
    mesh = plsc.VectorSubcoreMesh(
        core_axis_name="c", subcore_axis_name="s",
        num_cores=NC, num_subcores=NSUB)
    return pl.kernel(
        _sg_body,
        out_type=jax.ShapeDtypeStruct((NW, NSC, BPW), jnp.float32),
        mesh=mesh,
        compiler_params=pltpu.CompilerParams(needs_layout_passes=False),
        scratch_types=[
            pltpu.VMEM((NIDX, BPW), jnp.int32),
            pltpu.VMEM((BPW, D), jnp.float32),
            pltpu.VMEM((BPW, D), jnp.float32),
            pltpu.VMEM((NS, BPW, D), jnp.float32),
            pltpu.VMEM((3 * L * PPITCH,), jnp.float32),
            pltpu.VMEM((NSC, BPW), jnp.float32),
            pltpu.SemaphoreType.DMA,
            pltpu.SemaphoreType.DMA,
            pltpu.SemaphoreType.DMA,
            pltpu.SemaphoreType.DMA,
        ],
    )(idx_all, embeddings, context_embeddings)


def kernel(center_word, context_word, negative_samples, embeddings, context_embeddings):
    idx_all = jnp.concatenate(
        [center_word[None, :], context_word[None, :], negative_samples.T],
        axis=0)
    idx_all = idx_all.reshape(NIDX, NW, BPW).transpose(1, 0, 2)
    out = _skipgram(idx_all, embeddings, context_embeddings)
    pos_score = out[:, 0, :].reshape(B)
    neg_score = out[:, 1:, :].transpose(0, 2, 1).reshape(B, NS)
    return (pos_score, neg_score)
